# baseline (speedup 1.0000x reference)
import functools

import jax
import jax.numpy as jnp
from jax import lax
from jax.experimental import pallas as pl
from jax.experimental.pallas import tpu as pltpu
from jax.experimental.pallas import tpu_sc as plsc

N = 10000
NPAD = 10240
E = 320000
NC = 2
NS = 16
NW = NC * NS
B = 80
RPT = E // (NW * B)
EPT = RPT * B
NBUF = 5
NROUND = RPT // NBUF
NPT = NPAD // NS
ZR = 128

_MESH = dict(core_axis_name="c", subcore_axis_name="s")
_SC_PARAMS = pltpu.CompilerParams(use_tc_tiling_on_sc=False)


def _deg_call(edge_index):
  @functools.partial(
      pl.kernel,
      out_type=jax.ShapeDtypeStruct((NC * NPAD,), jnp.float32),
      mesh=plsc.VectorSubcoreMesh(**_MESH),
      compiler_params=_SC_PARAMS,
      scratch_types=[
          pltpu.VMEM((EPT,), jnp.int32),
          pltpu.VMEM((B,), jnp.float32),
          pltpu.VMEM((NPT,), jnp.float32),
          pltpu.VMEM_SHARED((NPAD,), jnp.float32),
          pltpu.SemaphoreType.DMA,
      ],
  )
  def deg_kernel(ei_hbm, out_hbm, didx, ones_v, zcol, acc, sem):
    cid = lax.axis_index("c")
    sid = lax.axis_index("s")
    wid = cid * NS + sid
    onev = jnp.full((16,), 1.0, jnp.float32)
    for j in range(B // 16):
      ones_v[pl.ds(j * 16, 16)] = onev

    def zbody(i, carry):
      zcol[pl.ds(i * 16, 16)] = jnp.zeros((16,), jnp.float32)
      return carry

    lax.fori_loop(0, NPT // 16, zbody, 0)
    pltpu.sync_copy(ei_hbm.at[1, pl.ds(wid * EPT, EPT)], didx)
    pltpu.sync_copy(zcol, acc.at[pl.ds(sid * NPT, NPT)])
    plsc.subcore_barrier()

    def body(j, carry):
      i0 = j * NBUF
      descs = [
          pltpu.async_copy(
              ones_v, acc.at[didx.at[pl.ds((i0 + k) * B, B)]], sem, add=True)
          for k in range(NBUF)
      ]
      for d in descs:
        d.wait()
      return carry

    lax.fori_loop(0, NROUND, body, 0)
    plsc.subcore_barrier()
    pltpu.sync_copy(acc.at[pl.ds(sid * NPT, NPT)],
                    out_hbm.at[pl.ds(cid * NPAD + sid * NPT, NPT)])

  return deg_kernel(edge_index)


def _make_scatter(D):
  @functools.partial(
      pl.kernel,
      out_type=jax.ShapeDtypeStruct((NC, NPAD, D), jnp.float32),
      mesh=plsc.VectorSubcoreMesh(**_MESH),
      compiler_params=_SC_PARAMS,
      scratch_types=[
          pltpu.VMEM((EPT,), jnp.int32),
          pltpu.VMEM((EPT,), jnp.int32),
          pltpu.VMEM((NBUF, B, D), jnp.float32),
          pltpu.VMEM((ZR, D), jnp.float32),
          pltpu.VMEM_SHARED((NPAD, D), jnp.float32),
          pltpu.SemaphoreType.DMA,
          pltpu.SemaphoreType.DMA,
      ],
  )
  def scatter_kernel(ei_hbm, g_hbm, out_hbm, sidx, didx, rows, zrow, acc,
                     gsem, ssem):
    cid = lax.axis_index("c")
    sid = lax.axis_index("s")
    wid = cid * NS + sid

    def zbody(i, carry):
      for j in range(D // 16):
        zrow[i, pl.ds(j * 16, 16)] = jnp.zeros((16,), jnp.float32)
      return carry

    lax.fori_loop(0, ZR, zbody, 0)
    pltpu.sync_copy(ei_hbm.at[0, pl.ds(wid * EPT, EPT)], sidx)
    pltpu.sync_copy(ei_hbm.at[1, pl.ds(wid * EPT, EPT)], didx)
    for t in range(NPT // ZR):
      pltpu.sync_copy(zrow, acc.at[pl.ds(sid * NPT + t * ZR, ZR)])
    plsc.subcore_barrier()

    def body(j, carry):
      i0 = j * NBUF
      gds = [
          pltpu.async_copy(
              g_hbm.at[sidx.at[pl.ds((i0 + k) * B, B)]], rows.at[k], gsem)
          for k in range(NBUF)
      ]
      for d in gds:
        d.wait()
      sds = [
          pltpu.async_copy(
              rows.at[k], acc.at[didx.at[pl.ds((i0 + k) * B, B)]], ssem,
              add=True)
          for k in range(NBUF)
      ]
      for d in sds:
        d.wait()
      return carry

    lax.fori_loop(0, NROUND, body, 0)
    plsc.subcore_barrier()
    pltpu.sync_copy(acc.at[pl.ds(sid * NPT, NPT)],
                    out_hbm.at[cid, pl.ds(sid * NPT, NPT)])

  return scatter_kernel


_R = 2000


def _tc_head_body(deg_ref, x_ref, w1_ref, g1_ref, dinv_ref):
  deg = jnp.sum(deg_ref[...], axis=1, keepdims=True) + 1.0
  dinv = lax.rsqrt(deg)
  hw = jnp.dot(x_ref[...], w1_ref[...], preferred_element_type=jnp.float32)
  g1_ref[...] = hw * dinv
  dinv_ref[...] = dinv


def _tc_head(degT, x, W1):
  return pl.pallas_call(
      _tc_head_body,
      grid=(N // _R,),
      in_specs=[
          pl.BlockSpec((_R, NC), lambda i: (i, 0)),
          pl.BlockSpec((_R, 128), lambda i: (i, 0)),
          pl.BlockSpec((128, 32), lambda i: (0, 0)),
      ],
      out_specs=[
          pl.BlockSpec((_R, 32), lambda i: (i, 0)),
          pl.BlockSpec((_R, 1), lambda i: (i, 0)),
      ],
      out_shape=[
          jax.ShapeDtypeStruct((NPAD, 32), jnp.float32),
          jax.ShapeDtypeStruct((N, 1), jnp.float32),
      ],
  )(degT, x, W1)


def _tc_mid_body(s_ref, g1_ref, dinv_ref, b1_ref, w2_ref, g2_ref):
  s = s_ref[0] + s_ref[1] + g1_ref[...]
  h = jnp.maximum(s * dinv_ref[...] + b1_ref[...], 0.0)
  g2_ref[...] = jnp.dot(h, w2_ref[...],
                        preferred_element_type=jnp.float32) * dinv_ref[...]


def _tc_mid(s1, g1, dinv, b1, W2):
  return pl.pallas_call(
      _tc_mid_body,
      grid=(N // _R,),
      in_specs=[
          pl.BlockSpec((NC, _R, 32), lambda i: (0, i, 0)),
          pl.BlockSpec((_R, 32), lambda i: (i, 0)),
          pl.BlockSpec((_R, 1), lambda i: (i, 0)),
          pl.BlockSpec((1, 32), lambda i: (0, 0)),
          pl.BlockSpec((32, 16), lambda i: (0, 0)),
      ],
      out_specs=pl.BlockSpec((_R, 16), lambda i: (i, 0)),
      out_shape=jax.ShapeDtypeStruct((NPAD, 16), jnp.float32),
  )(s1, g1, dinv, b1, W2)


def _tc_tail_body(s_ref, g2_ref, dinv_ref, b2_ref, out_ref):
  out_ref[...] = (s_ref[0] + s_ref[1] + g2_ref[...]) * dinv_ref[...] + b2_ref[...]


def _tc_tail(s2, g2, dinv, b2):
  return pl.pallas_call(
      _tc_tail_body,
      grid=(N // _R,),
      in_specs=[
          pl.BlockSpec((NC, _R, 16), lambda i: (0, i, 0)),
          pl.BlockSpec((_R, 16), lambda i: (i, 0)),
          pl.BlockSpec((_R, 1), lambda i: (i, 0)),
          pl.BlockSpec((1, 16), lambda i: (0, 0)),
      ],
      out_specs=pl.BlockSpec((_R, 16), lambda i: (i, 0)),
      out_shape=jax.ShapeDtypeStruct((N, 16), jnp.float32),
  )(s2, g2, dinv, b2)


def kernel(x, edge_index, W1, b1, W2, b2):
  degp = _deg_call(edge_index)
  degT = degp.reshape(NC, NPAD).T
  g1, dinv = _tc_head(degT, x, W1)
  s1 = _make_scatter(32)(edge_index, g1)
  g2 = _tc_mid(s1, g1, dinv, b1.reshape(1, 32), W2)
  s2 = _make_scatter(16)(edge_index, g2)
  return _tc_tail(s2, g2, dinv, b2.reshape(1, 16))


# speedup vs baseline: 1.1307x; 1.1307x over previous
import functools

import jax
import jax.numpy as jnp
from jax import lax
from jax.experimental import pallas as pl
from jax.experimental.pallas import tpu as pltpu
from jax.experimental.pallas import tpu_sc as plsc

N = 10000
NPAD = 10240
E = 320000
NC = 2
NS = 16
NW = NC * NS
B = 80
RPT = E // (NW * B)
EPT = RPT * B
NBUF = 25
NROUND = RPT // NBUF
NPT = NPAD // NS
ZR = 128

_MESH = dict(core_axis_name="c", subcore_axis_name="s")
_SC_PARAMS = pltpu.CompilerParams(use_tc_tiling_on_sc=False)


def _deg_call(edge_index):
  @functools.partial(
      pl.kernel,
      out_type=jax.ShapeDtypeStruct((NC * NPAD,), jnp.float32),
      mesh=plsc.VectorSubcoreMesh(**_MESH),
      compiler_params=_SC_PARAMS,
      scratch_types=[
          pltpu.VMEM((EPT,), jnp.int32),
          pltpu.VMEM((B,), jnp.float32),
          pltpu.VMEM((NPT,), jnp.float32),
          pltpu.VMEM_SHARED((NPAD,), jnp.float32),
          pltpu.SemaphoreType.DMA,
      ],
  )
  def deg_kernel(ei_hbm, out_hbm, didx, ones_v, zcol, acc, sem):
    cid = lax.axis_index("c")
    sid = lax.axis_index("s")
    wid = cid * NS + sid
    onev = jnp.full((16,), 1.0, jnp.float32)
    for j in range(B // 16):
      ones_v[pl.ds(j * 16, 16)] = onev

    def zbody(i, carry):
      zcol[pl.ds(i * 16, 16)] = jnp.zeros((16,), jnp.float32)
      return carry

    lax.fori_loop(0, NPT // 16, zbody, 0)
    pltpu.sync_copy(ei_hbm.at[1, pl.ds(wid * EPT, EPT)], didx)
    pltpu.sync_copy(zcol, acc.at[pl.ds(sid * NPT, NPT)])
    plsc.subcore_barrier()

    def body(j, carry):
      i0 = j * NBUF
      descs = [
          pltpu.async_copy(
              ones_v, acc.at[didx.at[pl.ds((i0 + k) * B, B)]], sem, add=True)
          for k in range(NBUF)
      ]
      for d in descs:
        d.wait()
      return carry

    lax.fori_loop(0, NROUND, body, 0)
    plsc.subcore_barrier()
    pltpu.sync_copy(acc.at[pl.ds(sid * NPT, NPT)],
                    out_hbm.at[pl.ds(cid * NPAD + sid * NPT, NPT)])

  return deg_kernel(edge_index)


def _make_scatter(D):
  @functools.partial(
      pl.kernel,
      out_type=jax.ShapeDtypeStruct((NC, NPAD, D), jnp.float32),
      mesh=plsc.VectorSubcoreMesh(**_MESH),
      compiler_params=_SC_PARAMS,
      scratch_types=[
          pltpu.VMEM((EPT,), jnp.int32),
          pltpu.VMEM((EPT,), jnp.int32),
          pltpu.VMEM((NBUF, B, D), jnp.float32),
          pltpu.VMEM((ZR, D), jnp.float32),
          pltpu.VMEM_SHARED((NPAD, D), jnp.float32),
          pltpu.SemaphoreType.DMA,
          pltpu.SemaphoreType.DMA,
      ],
  )
  def scatter_kernel(ei_hbm, g_hbm, out_hbm, sidx, didx, rows, zrow, acc,
                     gsem, ssem):
    cid = lax.axis_index("c")
    sid = lax.axis_index("s")
    wid = cid * NS + sid

    def zbody(i, carry):
      for j in range(D // 16):
        zrow[i, pl.ds(j * 16, 16)] = jnp.zeros((16,), jnp.float32)
      return carry

    lax.fori_loop(0, ZR, zbody, 0)
    pltpu.sync_copy(ei_hbm.at[0, pl.ds(wid * EPT, EPT)], sidx)
    pltpu.sync_copy(ei_hbm.at[1, pl.ds(wid * EPT, EPT)], didx)
    for t in range(NPT // ZR):
      pltpu.sync_copy(zrow, acc.at[pl.ds(sid * NPT + t * ZR, ZR)])
    plsc.subcore_barrier()

    def body(j, carry):
      i0 = j * NBUF
      gds = [
          pltpu.async_copy(
              g_hbm.at[sidx.at[pl.ds((i0 + k) * B, B)]], rows.at[k], gsem)
          for k in range(NBUF)
      ]
      for d in gds:
        d.wait()
      sds = [
          pltpu.async_copy(
              rows.at[k], acc.at[didx.at[pl.ds((i0 + k) * B, B)]], ssem,
              add=True)
          for k in range(NBUF)
      ]
      for d in sds:
        d.wait()
      return carry

    lax.fori_loop(0, NROUND, body, 0)
    plsc.subcore_barrier()
    pltpu.sync_copy(acc.at[pl.ds(sid * NPT, NPT)],
                    out_hbm.at[cid, pl.ds(sid * NPT, NPT)])

  return scatter_kernel


_R = 2000


def _dinv_of(deg_ref):
  deg = jnp.sum(deg_ref[...], axis=1, keepdims=True) + 1.0
  return lax.rsqrt(deg)


def _tc_head_body(deg_ref, x_ref, w1_ref, g1_ref):
  hw = jnp.dot(x_ref[...], w1_ref[...], preferred_element_type=jnp.float32)
  g1_ref[...] = hw * _dinv_of(deg_ref)


def _tc_head(degT, x, W1):
  return pl.pallas_call(
      _tc_head_body,
      grid=(N // _R,),
      in_specs=[
          pl.BlockSpec((_R, NC), lambda i: (i, 0)),
          pl.BlockSpec((_R, 128), lambda i: (i, 0)),
          pl.BlockSpec((128, 32), lambda i: (0, 0)),
      ],
      out_specs=pl.BlockSpec((_R, 32), lambda i: (i, 0)),
      out_shape=jax.ShapeDtypeStruct((NPAD, 32), jnp.float32),
  )(degT, x, W1)


def _tc_mid_body(s_ref, g1_ref, deg_ref, b1_ref, w2_ref, g2_ref):
  dinv = _dinv_of(deg_ref)
  s = s_ref[0] + s_ref[1] + g1_ref[...]
  h = jnp.maximum(s * dinv + b1_ref[...], 0.0)
  g2_ref[...] = jnp.dot(h, w2_ref[...],
                        preferred_element_type=jnp.float32) * dinv


def _tc_mid(s1, g1, degT, b1, W2):
  return pl.pallas_call(
      _tc_mid_body,
      grid=(N // _R,),
      in_specs=[
          pl.BlockSpec((NC, _R, 32), lambda i: (0, i, 0)),
          pl.BlockSpec((_R, 32), lambda i: (i, 0)),
          pl.BlockSpec((_R, NC), lambda i: (i, 0)),
          pl.BlockSpec((1, 32), lambda i: (0, 0)),
          pl.BlockSpec((32, 16), lambda i: (0, 0)),
      ],
      out_specs=pl.BlockSpec((_R, 16), lambda i: (i, 0)),
      out_shape=jax.ShapeDtypeStruct((NPAD, 16), jnp.float32),
  )(s1, g1, degT, b1, W2)


def _tc_tail_body(s_ref, g2_ref, deg_ref, b2_ref, out_ref):
  out_ref[...] = ((s_ref[0] + s_ref[1] + g2_ref[...]) * _dinv_of(deg_ref)
                  + b2_ref[...])


def _tc_tail(s2, g2, degT, b2):
  return pl.pallas_call(
      _tc_tail_body,
      grid=(N // _R,),
      in_specs=[
          pl.BlockSpec((NC, _R, 16), lambda i: (0, i, 0)),
          pl.BlockSpec((_R, 16), lambda i: (i, 0)),
          pl.BlockSpec((_R, NC), lambda i: (i, 0)),
          pl.BlockSpec((1, 16), lambda i: (0, 0)),
      ],
      out_specs=pl.BlockSpec((_R, 16), lambda i: (i, 0)),
      out_shape=jax.ShapeDtypeStruct((N, 16), jnp.float32),
  )(s2, g2, degT, b2)


def kernel(x, edge_index, W1, b1, W2, b2):
  degp = _deg_call(edge_index)
  degT = degp.reshape(NC, NPAD).T
  g1 = _tc_head(degT, x, W1)
  s1 = _make_scatter(32)(edge_index, g1)
  g2 = _tc_mid(s1, g1, degT, b1.reshape(1, 32), W2)
  s2 = _make_scatter(16)(edge_index, g2)
  return _tc_tail(s2, g2, degT, b2.reshape(1, 16))


# speedup vs baseline: 1.3749x; 1.2160x over previous
import functools

import jax
import jax.numpy as jnp
from jax import lax
from jax.experimental import pallas as pl
from jax.experimental.pallas import tpu as pltpu
from jax.experimental.pallas import tpu_sc as plsc

N = 10000
NPAD = 10240
E = 320000
NC = 2
NS = 16
NW = NC * NS
B = 80
RPT = E // (NW * B)
EPT = RPT * B
NBUF = 25
NROUND = RPT // NBUF
NPT = NPAD // NS
ZR = 128

_MESH = dict(core_axis_name="c", subcore_axis_name="s")
_SC_PARAMS = pltpu.CompilerParams(use_tc_tiling_on_sc=False)


def _deg_call(edge_index):
  @functools.partial(
      pl.kernel,
      out_type=jax.ShapeDtypeStruct((NC * NPAD,), jnp.float32),
      mesh=plsc.VectorSubcoreMesh(**_MESH),
      compiler_params=_SC_PARAMS,
      scratch_types=[
          pltpu.VMEM((EPT,), jnp.int32),
          pltpu.VMEM((B,), jnp.float32),
          pltpu.VMEM((NPT,), jnp.float32),
          pltpu.VMEM_SHARED((NPAD,), jnp.float32),
          pltpu.SemaphoreType.DMA,
      ],
  )
  def deg_kernel(ei_hbm, out_hbm, didx, ones_v, zcol, acc, sem):
    cid = lax.axis_index("c")
    sid = lax.axis_index("s")
    wid = cid * NS + sid
    onev = jnp.full((16,), 1.0, jnp.float32)
    for j in range(B // 16):
      ones_v[pl.ds(j * 16, 16)] = onev

    def zbody(i, carry):
      zcol[pl.ds(i * 16, 16)] = jnp.zeros((16,), jnp.float32)
      return carry

    lax.fori_loop(0, NPT // 16, zbody, 0)
    pltpu.sync_copy(ei_hbm.at[1, pl.ds(wid * EPT, EPT)], didx)
    pltpu.sync_copy(zcol, acc.at[pl.ds(sid * NPT, NPT)])
    plsc.subcore_barrier()

    def body(j, carry):
      i0 = j * NBUF
      descs = [
          pltpu.async_copy(
              ones_v, acc.at[didx.at[pl.ds((i0 + k) * B, B)]], sem, add=True)
          for k in range(NBUF)
      ]
      for d in descs:
        d.wait()
      return carry

    lax.fori_loop(0, NROUND, body, 0)
    plsc.subcore_barrier()
    pltpu.sync_copy(acc.at[pl.ds(sid * NPT, NPT)],
                    out_hbm.at[pl.ds(cid * NPAD + sid * NPT, NPT)])

  return deg_kernel(edge_index)


def _make_scatter(D):
  @functools.partial(
      pl.kernel,
      out_type=jax.ShapeDtypeStruct((NC, NPAD, D), jnp.float32),
      mesh=plsc.VectorSubcoreMesh(**_MESH),
      compiler_params=_SC_PARAMS,
      scratch_types=[
          pltpu.VMEM((EPT,), jnp.int32),
          pltpu.VMEM((EPT,), jnp.int32),
          pltpu.VMEM((NBUF, B, D), jnp.float32),
          pltpu.VMEM((ZR, D), jnp.float32),
          pltpu.VMEM_SHARED((NPAD, D), jnp.float32),
          pltpu.SemaphoreType.DMA,
          pltpu.SemaphoreType.DMA,
      ],
  )
  def scatter_kernel(ei_hbm, g_hbm, out_hbm, sidx, didx, rows, zrow, acc,
                     gsem, ssem):
    cid = lax.axis_index("c")
    sid = lax.axis_index("s")
    wid = cid * NS + sid

    def zbody(i, carry):
      for j in range(D // 16):
        zrow[i, pl.ds(j * 16, 16)] = jnp.zeros((16,), jnp.float32)
      return carry

    lax.fori_loop(0, ZR, zbody, 0)
    pltpu.sync_copy(ei_hbm.at[0, pl.ds(wid * EPT, EPT)], sidx)
    pltpu.sync_copy(ei_hbm.at[1, pl.ds(wid * EPT, EPT)], didx)
    for t in range(NPT // ZR):
      pltpu.sync_copy(zrow, acc.at[pl.ds(sid * NPT + t * ZR, ZR)])
    plsc.subcore_barrier()

    def body(j, carry):
      i0 = j * NBUF
      gds = [
          pltpu.async_copy(
              g_hbm.at[sidx.at[pl.ds((i0 + k) * B, B)]], rows.at[k], gsem)
          for k in range(NBUF)
      ]
      for d in gds:
        d.wait()
      sds = [
          pltpu.async_copy(
              rows.at[k], acc.at[didx.at[pl.ds((i0 + k) * B, B)]], ssem,
              add=True)
          for k in range(NBUF)
      ]
      for d in sds:
        d.wait()
      return carry

    lax.fori_loop(0, NROUND, body, 0)
    plsc.subcore_barrier()
    pltpu.sync_copy(acc.at[pl.ds(sid * NPT, NPT)],
                    out_hbm.at[cid, pl.ds(sid * NPT, NPT)])

  return scatter_kernel


_RB = 512
_GRID = NPAD // 4 // _RB


def _rep(npack):
  lane = lax.broadcasted_iota(jnp.int32, (npack, 128), 1)
  grp = lax.broadcasted_iota(jnp.int32, (npack, 128), 0)
  return jnp.where(lane // (128 // npack) == grp, 1.0, 0.0).astype(jnp.float32)


def _tc_head_body(deg_ref, x4_ref, w1_ref, g1_ref):
  dinv = lax.rsqrt(deg_ref[...] + 1.0)
  dinv4 = jnp.dot(dinv, _rep(4), preferred_element_type=jnp.float32)
  hw4 = jnp.dot(x4_ref[...], w1_ref[...], preferred_element_type=jnp.float32)
  g1_ref[...] = hw4 * dinv4


def _tc_head(deg4, x4, W1blk):
  return pl.pallas_call(
      _tc_head_body,
      grid=(_GRID,),
      in_specs=[
          pl.BlockSpec((_RB, 4), lambda i: (i, 0)),
          pl.BlockSpec((_RB, 512), lambda i: (i, 0)),
          pl.BlockSpec((512, 128), lambda i: (0, 0)),
      ],
      out_specs=pl.BlockSpec((_RB, 128), lambda i: (i, 0)),
      out_shape=jax.ShapeDtypeStruct((NPAD // 4, 128), jnp.float32),
  )(deg4, x4, W1blk)


def _tc_mid_body(s_ref, g1_ref, deg_ref, b1_ref, w2_ref, g2_ref):
  dinv = lax.rsqrt(deg_ref[...] + 1.0)
  dinv4 = jnp.dot(dinv, _rep(4), preferred_element_type=jnp.float32)
  s = s_ref[0] + s_ref[1] + g1_ref[...]
  h = jnp.maximum(s * dinv4 + b1_ref[...], 0.0)
  g2_ref[...] = jnp.dot(h * dinv4, w2_ref[...],
                        preferred_element_type=jnp.float32)


def _tc_mid(s1p, g1p, deg4, b1p, W2blk):
  return pl.pallas_call(
      _tc_mid_body,
      grid=(_GRID,),
      in_specs=[
          pl.BlockSpec((NC, _RB, 128), lambda i: (0, i, 0)),
          pl.BlockSpec((_RB, 128), lambda i: (i, 0)),
          pl.BlockSpec((_RB, 4), lambda i: (i, 0)),
          pl.BlockSpec((1, 128), lambda i: (0, 0)),
          pl.BlockSpec((128, 64), lambda i: (0, 0)),
      ],
      out_specs=pl.BlockSpec((_RB, 64), lambda i: (i, 0)),
      out_shape=jax.ShapeDtypeStruct((NPAD // 4, 64), jnp.float32),
  )(s1p, g1p, deg4, b1p, W2blk)


def _tc_tail_body(s_ref, g2_ref, deg_ref, b2_ref, out_ref):
  dinv = lax.rsqrt(deg_ref[...] + 1.0)
  dinv8 = jnp.dot(dinv, _rep(8), preferred_element_type=jnp.float32)
  s = s_ref[0] + s_ref[1] + g2_ref[...]
  out_ref[...] = s * dinv8 + b2_ref[...]


def _tc_tail(s2p, g2p8, deg8, b2p):
  rb2 = _RB // 2
  return pl.pallas_call(
      _tc_tail_body,
      grid=(_GRID,),
      in_specs=[
          pl.BlockSpec((NC, rb2, 128), lambda i: (0, i, 0)),
          pl.BlockSpec((rb2, 128), lambda i: (i, 0)),
          pl.BlockSpec((rb2, 8), lambda i: (i, 0)),
          pl.BlockSpec((1, 128), lambda i: (0, 0)),
      ],
      out_specs=pl.BlockSpec((rb2, 128), lambda i: (i, 0)),
      out_shape=jax.ShapeDtypeStruct((NPAD // 8, 128), jnp.float32),
  )(s2p, g2p8, deg8, b2p)


def kernel(x, edge_index, W1, b1, W2, b2):
  W1blk = jnp.kron(jnp.eye(4, dtype=jnp.float32), W1)
  W2blk = jnp.kron(jnp.eye(4, dtype=jnp.float32), W2)
  b1p = jnp.tile(b1, 4).reshape(1, 128)
  b2p = jnp.tile(b2, 8).reshape(1, 128)
  x4 = x.reshape(N // 4, 512)

  degp = _deg_call(edge_index)
  deg = degp[:NPAD] + degp[NPAD:]
  deg4 = deg.reshape(NPAD // 4, 4)
  deg8 = deg.reshape(NPAD // 8, 8)

  g1p = _tc_head(deg4, x4, W1blk)
  s1 = _make_scatter(32)(edge_index, g1p.reshape(NPAD, 32))
  s1p = s1.reshape(NC, NPAD // 4, 128)
  g2p = _tc_mid(s1p, g1p, deg4, b1p, W2blk)
  g2f = g2p.reshape(NPAD, 16)
  s2 = _make_scatter(16)(edge_index, g2f)
  s2p = s2.reshape(NC, NPAD // 8, 128)
  outp = _tc_tail(s2p, g2f.reshape(NPAD // 8, 128), deg8, b2p)
  return outp.reshape(NPAD, 16)[:N]
